# baseline (speedup 1.0000x reference)
import jax
import jax.numpy as jnp
from jax import lax
from jax.experimental import pallas as pl
from jax.experimental.pallas import tpu as pltpu
from jax.experimental.pallas import tpu_sc as plsc

_NC = 2
_NS = 16
_NW = _NC * _NS
_L = 16

_BATCH = 16384
_EMB_DIM = 16
_TC_N = 7168
_SC_N = _BATCH - _TC_N
_B_PER_W = _SC_N // _NW


def _sc_body(y_hbm, table_hbm, out_hbm, idx_v, rows_v, sem):
    wid = lax.axis_index("s") * _NC + lax.axis_index("c")
    base = wid * _B_PER_W
    pltpu.sync_copy(y_hbm.at[pl.ds(base, _B_PER_W)], idx_v)

    def issue(g, _):
        vec = idx_v[pl.ds(g * _L, _L)]
        for lane in range(_L):
            pltpu.make_async_copy(
                table_hbm.at[pl.ds(vec[lane], 1)],
                rows_v.at[pl.ds(g * _L + lane, 1)],
                sem,
            ).start()
        return ()

    lax.fori_loop(0, _B_PER_W // _L, issue, ())
    pltpu.make_async_copy(table_hbm.at[pl.ds(0, _B_PER_W)], rows_v, sem).wait()
    pltpu.sync_copy(rows_v, out_hbm.at[pl.ds(base, _B_PER_W)])


def _tc_body(idx_smem, table_hbm, o_ref, sem):
    def it(i, _):
        s = idx_smem[i]
        pltpu.make_async_copy(
            table_hbm.at[pl.ds(s, 1)], o_ref.at[pl.ds(i, 1)], sem
        ).start()
        return ()

    lax.fori_loop(0, _TC_N, it, (), unroll=8)
    pltpu.make_async_copy(table_hbm.at[pl.ds(0, _TC_N)], o_ref, sem).wait()


@jax.jit
def _gather(y, emb_table):
    mesh = plsc.VectorSubcoreMesh(core_axis_name="c", subcore_axis_name="s")
    sc_kern = pl.kernel(
        _sc_body,
        out_type=jax.ShapeDtypeStruct((_SC_N, _EMB_DIM), jnp.float32),
        mesh=mesh,
        scratch_types=[
            pltpu.VMEM((_B_PER_W,), jnp.int32),
            pltpu.VMEM((_B_PER_W, _EMB_DIM), jnp.float32),
            pltpu.SemaphoreType.DMA,
        ],
    )
    tc_kern = pl.pallas_call(
        _tc_body,
        grid_spec=pltpu.PrefetchScalarGridSpec(
            num_scalar_prefetch=1,
            grid=(1,),
            in_specs=[pl.BlockSpec(memory_space=pltpu.MemorySpace.HBM)],
            out_specs=pl.BlockSpec((_TC_N, _EMB_DIM), lambda i, idx: (0, 0)),
            scratch_shapes=[pltpu.SemaphoreType.DMA],
        ),
        out_shape=jax.ShapeDtypeStruct((_TC_N, _EMB_DIM), jnp.float32),
    )
    out_sc = sc_kern(y[_TC_N:], emb_table)
    out_tc = tc_kern(y[:_TC_N], emb_table)
    return jnp.concatenate([out_tc, out_sc], axis=0)


def kernel(y, emb_table):
    return _gather(y.astype(jnp.int32), emb_table)


# speedup vs baseline: 1.0976x; 1.0976x over previous
import jax
import jax.numpy as jnp
from jax import lax
from jax.experimental import pallas as pl
from jax.experimental.pallas import tpu as pltpu
from jax.experimental.pallas import tpu_sc as plsc

_NC = 2
_NS = 16
_NW = _NC * _NS
_L = 16

_BATCH = 16384
_EMB_DIM = 16
_B_PER_W = _BATCH // _NW


def _gather_body(y_hbm, table_hbm, out_hbm, idx_v, rows_v, sem):
    wid = lax.axis_index("s") * _NC + lax.axis_index("c")
    base = wid * _B_PER_W
    pltpu.sync_copy(y_hbm.at[pl.ds(base, _B_PER_W)], idx_v)

    def issue(g, _):
        vec = idx_v[pl.ds(g * _L, _L)]
        for lane in range(_L):
            pltpu.make_async_copy(
                table_hbm.at[pl.ds(vec[lane], 1)],
                rows_v.at[pl.ds(g * _L + lane, 1)],
                sem,
            ).start()
        return ()

    lax.fori_loop(0, _B_PER_W // _L, issue, ())
    pltpu.make_async_copy(table_hbm.at[pl.ds(0, _B_PER_W)], rows_v, sem).wait()
    pltpu.sync_copy(rows_v, out_hbm.at[pl.ds(base, _B_PER_W)])


@jax.jit
def _gather(y, emb_table):
    mesh = plsc.VectorSubcoreMesh(core_axis_name="c", subcore_axis_name="s")
    kern = pl.kernel(
        _gather_body,
        out_type=jax.ShapeDtypeStruct((_BATCH, _EMB_DIM), jnp.float32),
        mesh=mesh,
        scratch_types=[
            pltpu.VMEM((_B_PER_W,), jnp.int32),
            pltpu.VMEM((_B_PER_W, _EMB_DIM), jnp.float32),
            pltpu.SemaphoreType.DMA,
        ],
    )
    return kern(y, emb_table)


def kernel(y, emb_table):
    return _gather(y.astype(jnp.int32), emb_table)


# speedup vs baseline: 1.1043x; 1.0061x over previous
import jax
import jax.numpy as jnp
from jax import lax
from jax.experimental import pallas as pl
from jax.experimental.pallas import tpu as pltpu
from jax.experimental.pallas import tpu_sc as plsc

_NC = 2
_NS = 16
_NW = _NC * _NS
_L = 16

_BATCH = 16384
_EMB_DIM = 16
_B_PER_W = _BATCH // _NW
_NQ = 4
_Q = _B_PER_W // _NQ


def _gather_body(y_hbm, table_hbm, out_hbm, idx_v, rows_v, osem, sems):
    wid = lax.axis_index("s") * _NC + lax.axis_index("c")
    base = wid * _B_PER_W
    pltpu.sync_copy(y_hbm.at[pl.ds(base, _B_PER_W)], idx_v)

    def issue(q):
        def grp(g, _):
            i0 = q * _Q + g * _L
            vec = idx_v[pl.ds(i0, _L)]
            for lane in range(_L):
                pltpu.make_async_copy(
                    table_hbm.at[pl.ds(vec[lane], 1)],
                    rows_v.at[pl.ds(i0 + lane, 1)],
                    sems.at[q],
                ).start()
            return ()

        lax.fori_loop(0, _Q // _L, grp, ())

    for q in range(_NQ):
        issue(q)
    for q in range(_NQ):
        pltpu.make_async_copy(
            table_hbm.at[pl.ds(0, _Q)],
            rows_v.at[pl.ds(0, _Q)],
            sems.at[q],
        ).wait()
        pltpu.make_async_copy(
            rows_v.at[pl.ds(q * _Q, _Q)],
            out_hbm.at[pl.ds(base + q * _Q, _Q)],
            osem,
        ).start()
    for q in range(_NQ):
        pltpu.make_async_copy(
            rows_v.at[pl.ds(0, _Q)],
            out_hbm.at[pl.ds(base, _Q)],
            osem,
        ).wait()


@jax.jit
def _gather(y, emb_table):
    mesh = plsc.VectorSubcoreMesh(core_axis_name="c", subcore_axis_name="s")
    kern = pl.kernel(
        _gather_body,
        out_type=jax.ShapeDtypeStruct((_BATCH, _EMB_DIM), jnp.float32),
        mesh=mesh,
        scratch_types=[
            pltpu.VMEM((_B_PER_W,), jnp.int32),
            pltpu.VMEM((_B_PER_W, _EMB_DIM), jnp.float32),
            pltpu.SemaphoreType.DMA,
            pltpu.SemaphoreType.DMA((_NQ,)),
        ],
    )
    return kern(y, emb_table)


def kernel(y, emb_table):
    return _gather(y.astype(jnp.int32), emb_table)
